# Initial kernel scaffold; baseline (speedup 1.0000x reference)
import jax
import jax.numpy as jnp
from jax.experimental import pallas as pl


def kernel(x, edge_index, batch, W1, b1, W2, b2, W3, b3, W4, b4, Wl, bl):
    raise NotImplementedError("write your pallas kernel here")


# speedup vs baseline: 7.6901x; 7.6901x over previous
import functools

import jax
import jax.numpy as jnp
from jax import lax
from jax.experimental import pallas as pl
from jax.experimental.pallas import tpu as pltpu
from jax.experimental.pallas import tpu_sc as plsc

N = 10000
E = 320000
D = 128
H = 128
C = 2
G = 512

N_PAD = 10240
G_PAD = 520
NW = 32
NCHUNK = 80
CHUNK = 128
E_PAD = NW * NCHUNK * CHUNK
ROWS_PT = N_PAD // 16
BR = 1024
BRP = 512


def _sc_mesh():
    return plsc.VectorSubcoreMesh(core_axis_name="c", subcore_axis_name="s")


def _sc_degree(dst_r, zeros1):
    @functools.partial(
        pl.kernel,
        out_type=jax.ShapeDtypeStruct((2 * N_PAD,), jnp.float32),
        mesh=_sc_mesh(),
        scratch_types=[
            pltpu.VMEM((NCHUNK, CHUNK), jnp.int32),
            pltpu.VMEM((CHUNK,), jnp.float32),
            pltpu.VMEM_SHARED((N_PAD,), jnp.float32),
        ],
    )
    def run(dst_hbm, z_hbm, deg_out, d_idx, ones_v, acc):
        cid = lax.axis_index("c")
        sid = lax.axis_index("s")
        wid = sid * 2 + cid
        pltpu.sync_copy(dst_hbm.at[pl.ds(wid * NCHUNK, NCHUNK)], d_idx)
        for i in range(CHUNK // 16):
            ones_v[pl.ds(i * 16, 16)] = jnp.ones((16,), jnp.float32)
        pltpu.sync_copy(z_hbm.at[pl.ds(sid * ROWS_PT, ROWS_PT)],
                        acc.at[pl.ds(sid * ROWS_PT, ROWS_PT)])
        plsc.subcore_barrier()

        def body(j, c):
            pltpu.sync_copy(ones_v, acc.at[d_idx.at[j]], add=True)
            return c

        lax.fori_loop(0, NCHUNK, body, 0)
        plsc.subcore_barrier()
        pltpu.sync_copy(acc.at[pl.ds(sid * ROWS_PT, ROWS_PT)],
                        deg_out.at[pl.ds(cid * N_PAD + sid * ROWS_PT, ROWS_PT)])

    return run(dst_r, zeros1)


PC = NCHUNK // 2


def _sc_scatter(p, src_r, dst_r, zeros2):
    @functools.partial(
        pl.kernel,
        out_type=jax.ShapeDtypeStruct((2 * N_PAD, H), jnp.float32),
        mesh=_sc_mesh(),
        scratch_types=[
            pltpu.VMEM((PC, CHUNK), jnp.int32),
            pltpu.VMEM((PC, CHUNK), jnp.int32),
            pltpu.VMEM((CHUNK, H), jnp.float32),
            pltpu.VMEM((CHUNK, H), jnp.float32),
            pltpu.VMEM_SHARED((N_PAD, H), jnp.float32),
            pltpu.SemaphoreType.DMA,
            pltpu.SemaphoreType.DMA,
        ],
    )
    def run(p_hbm, src_hbm, dst_hbm, z_hbm, s_out,
            s_idx, d_idx, buf0, buf1, acc, sem0, sem1):
        cid = lax.axis_index("c")
        sid = lax.axis_index("s")
        wid = sid * 2 + cid
        pltpu.sync_copy(z_hbm.at[pl.ds(sid * ROWS_PT, ROWS_PT)],
                        acc.at[pl.ds(sid * ROWS_PT, ROWS_PT)])
        plsc.subcore_barrier()

        for ph in range(2):
            base = wid * NCHUNK + ph * PC
            pltpu.sync_copy(src_hbm.at[pl.ds(base, PC)], s_idx)
            pltpu.sync_copy(dst_hbm.at[pl.ds(base, PC)], d_idx)

            pltpu.async_copy(p_hbm.at[s_idx.at[0]], buf0, sem0)
            pltpu.async_copy(p_hbm.at[s_idx.at[1]], buf1, sem1)

            def body(g, c):
                j0 = 2 * g
                j1 = j0 + 1
                pltpu.make_async_copy(p_hbm.at[s_idx.at[j0]], buf0, sem0).wait()
                pltpu.sync_copy(buf0, acc.at[d_idx.at[j0]], add=True)

                @pl.when(j0 + 2 < PC)
                def _():
                    pltpu.async_copy(p_hbm.at[s_idx.at[j0 + 2]], buf0, sem0)

                pltpu.make_async_copy(p_hbm.at[s_idx.at[j1]], buf1, sem1).wait()
                pltpu.sync_copy(buf1, acc.at[d_idx.at[j1]], add=True)

                @pl.when(j1 + 2 < PC)
                def _():
                    pltpu.async_copy(p_hbm.at[s_idx.at[j1 + 2]], buf1, sem1)

                return c

            lax.fori_loop(0, PC // 2, body, 0)

        plsc.subcore_barrier()
        pltpu.sync_copy(acc.at[pl.ds(sid * ROWS_PT, ROWS_PT)],
                        s_out.at[pl.ds(cid * N_PAD + sid * ROWS_PT, ROWS_PT)])

    return run(p, src_r, dst_r, zeros2)


def _tc_first(dega, degb, x, W1):
    def body(dega_ref, degb_ref, x_ref, w_ref, p_ref, dinv_ref):
        deg = dega_ref[...] + degb_ref[...] + 1.0
        dinv = lax.rsqrt(deg)
        dinv_ref[...] = dinv
        p_ref[...] = dinv * jnp.dot(x_ref[...], w_ref[...],
                                    preferred_element_type=jnp.float32)

    nb = N_PAD // BR
    return pl.pallas_call(
        body,
        grid=(nb,),
        in_specs=[
            pl.BlockSpec((BR, 1), lambda i: (i, 0)),
            pl.BlockSpec((BR, 1), lambda i: (i, 0)),
            pl.BlockSpec((BR, D), lambda i: (i, 0)),
            pl.BlockSpec((D, H), lambda i: (0, 0)),
        ],
        out_specs=[
            pl.BlockSpec((BR, H), lambda i: (i, 0)),
            pl.BlockSpec((BR, 1), lambda i: (i, 0)),
        ],
        out_shape=[
            jax.ShapeDtypeStruct((N_PAD, H), jnp.float32),
            jax.ShapeDtypeStruct((N_PAD, 1), jnp.float32),
        ],
    )(dega, degb, x, W1)


def _tc_mid(sa, sb, pp, dinv, b2d, W):
    def body(sa_ref, sb_ref, pp_ref, dinv_ref, b_ref, w_ref, p_ref):
        dinv = dinv_ref[...]
        h = jnp.maximum(dinv * (sa_ref[...] + sb_ref[...] + pp_ref[...])
                        + b_ref[...], 0.0)
        p_ref[...] = dinv * jnp.dot(h, w_ref[...],
                                    preferred_element_type=jnp.float32)

    nb = N_PAD // BR
    return pl.pallas_call(
        body,
        grid=(nb,),
        in_specs=[
            pl.BlockSpec((BR, H), lambda i: (i, 0)),
            pl.BlockSpec((BR, H), lambda i: (i, 0)),
            pl.BlockSpec((BR, H), lambda i: (i, 0)),
            pl.BlockSpec((BR, 1), lambda i: (i, 0)),
            pl.BlockSpec((1, H), lambda i: (0, 0)),
            pl.BlockSpec((H, H), lambda i: (0, 0)),
        ],
        out_specs=pl.BlockSpec((BR, H), lambda i: (i, 0)),
        out_shape=jax.ShapeDtypeStruct((N_PAD, H), jnp.float32),
    )(sa, sb, pp, dinv, b2d, W)


def _tc_pool(sa, sb, p4, dinv, b2d, batch3):
    def body(sa_ref, sb_ref, p_ref, dinv_ref, b_ref, batch_ref,
             pooled_ref, counts_ref):
        i = pl.program_id(0)
        h = dinv_ref[...] * (sa_ref[...] + sb_ref[...] + p_ref[...]) + b_ref[...]
        bb = batch_ref[0, 0, :]
        seg = lax.broadcasted_iota(jnp.int32, (G_PAD, BRP), 0)
        oh = (seg == bb[None, :]).astype(jnp.float32)

        @pl.when(i == 0)
        def _():
            pooled_ref[...] = jnp.zeros_like(pooled_ref)
            counts_ref[...] = jnp.zeros_like(counts_ref)

        pooled_ref[...] += jnp.dot(oh, h, preferred_element_type=jnp.float32)
        counts_ref[...] += jnp.sum(oh, axis=1, keepdims=True)

    nb = N_PAD // BRP
    return pl.pallas_call(
        body,
        grid=(nb,),
        in_specs=[
            pl.BlockSpec((BRP, H), lambda i: (i, 0)),
            pl.BlockSpec((BRP, H), lambda i: (i, 0)),
            pl.BlockSpec((BRP, H), lambda i: (i, 0)),
            pl.BlockSpec((BRP, 1), lambda i: (i, 0)),
            pl.BlockSpec((1, H), lambda i: (0, 0)),
            pl.BlockSpec((1, 1, BRP), lambda i: (i, 0, 0)),
        ],
        out_specs=[
            pl.BlockSpec((G_PAD, H), lambda i: (0, 0)),
            pl.BlockSpec((G_PAD, 1), lambda i: (0, 0)),
        ],
        out_shape=[
            jax.ShapeDtypeStruct((G_PAD, H), jnp.float32),
            jax.ShapeDtypeStruct((G_PAD, 1), jnp.float32),
        ],
    )(sa, sb, p4, dinv, b2d, batch3)


def _tc_final(pooled, counts, Wl_pad, bl_pad):
    def body(pooled_ref, counts_ref, wl_ref, bl_ref, out_ref):
        avg = pooled_ref[...] / jnp.maximum(counts_ref[...], 1.0)
        out_ref[...] = jnp.dot(avg, wl_ref[...],
                               preferred_element_type=jnp.float32) + bl_ref[...]

    return pl.pallas_call(
        body,
        out_shape=jax.ShapeDtypeStruct((G_PAD, H), jnp.float32),
    )(pooled, counts, Wl_pad, bl_pad)


def kernel(x, edge_index, batch, W1, b1, W2, b2, W3, b3, W4, b4, Wl, bl):
    f32 = jnp.float32
    src = edge_index[0].astype(jnp.int32)
    dst = edge_index[1].astype(jnp.int32)
    pad_e = E_PAD - E
    src_r = jnp.concatenate([src, jnp.full((pad_e,), N, jnp.int32)]
                            ).reshape(NW * NCHUNK, CHUNK)
    dst_r = jnp.concatenate([dst, jnp.full((pad_e,), N, jnp.int32)]
                            ).reshape(NW * NCHUNK, CHUNK)

    x_pad = jnp.concatenate([x.astype(f32), jnp.zeros((N_PAD - N, D), f32)])
    batch3 = jnp.concatenate([batch.astype(jnp.int32),
                              jnp.full((N_PAD - N,), G, jnp.int32)]
                             ).reshape(N_PAD // BRP, 1, BRP)
    zeros1 = jnp.zeros((N_PAD,), f32)
    zeros2 = jnp.zeros((N_PAD, H), f32)

    b1r = b1.reshape(1, H).astype(f32)
    b2r = b2.reshape(1, H).astype(f32)
    b3r = b3.reshape(1, H).astype(f32)
    b4r = b4.reshape(1, H).astype(f32)
    Wl_pad = jnp.pad(Wl.astype(f32), ((0, 0), (0, H - C)))
    bl_pad = jnp.pad(bl.astype(f32), (0, H - C)).reshape(1, H)

    deg_pair = _sc_degree(dst_r, zeros1).reshape(2, N_PAD)
    dega = deg_pair[0].reshape(N_PAD, 1)
    degb = deg_pair[1].reshape(N_PAD, 1)

    p1, dinv = _tc_first(dega, degb, x_pad, W1.astype(f32))
    s1 = _sc_scatter(p1, src_r, dst_r, zeros2).reshape(2, N_PAD, H)
    p2 = _tc_mid(s1[0], s1[1], p1, dinv, b1r, W2.astype(f32))
    s2 = _sc_scatter(p2, src_r, dst_r, zeros2).reshape(2, N_PAD, H)
    p3 = _tc_mid(s2[0], s2[1], p2, dinv, b2r, W3.astype(f32))
    s3 = _sc_scatter(p3, src_r, dst_r, zeros2).reshape(2, N_PAD, H)
    p4 = _tc_mid(s3[0], s3[1], p3, dinv, b3r, W4.astype(f32))
    s4 = _sc_scatter(p4, src_r, dst_r, zeros2).reshape(2, N_PAD, H)
    pooled, counts = _tc_pool(s4[0], s4[1], p4, dinv, b4r, batch3)
    out = _tc_final(pooled, counts, Wl_pad, bl_pad)
    return out[:G, :C]


# speedup vs baseline: 23.4166x; 3.0450x over previous
import functools

import jax
import jax.numpy as jnp
from jax import lax
from jax.experimental import pallas as pl
from jax.experimental.pallas import tpu as pltpu
from jax.experimental.pallas import tpu_sc as plsc

N = 10000
E = 320000
D = 128
H = 128
C = 2
G = 512

N_PAD = 10240
G_PAD = 520
NW = 32
NCHUNK = 80
CHUNK = 128
E_PAD = NW * NCHUNK * CHUNK
ROWS_PT = N_PAD // 16
BR = 1024
BRP = 512


def _sc_mesh():
    return plsc.VectorSubcoreMesh(core_axis_name="c", subcore_axis_name="s")


def _sc_degree(dst_r, zeros1):
    @functools.partial(
        pl.kernel,
        out_type=jax.ShapeDtypeStruct((2 * N_PAD,), jnp.float32),
        mesh=_sc_mesh(),
        scratch_types=[
            pltpu.VMEM((NCHUNK, CHUNK), jnp.int32),
            pltpu.VMEM((CHUNK,), jnp.float32),
            pltpu.VMEM_SHARED((N_PAD,), jnp.float32),
        ],
    )
    def run(dst_hbm, z_hbm, deg_out, d_idx, ones_v, acc):
        cid = lax.axis_index("c")
        sid = lax.axis_index("s")
        wid = sid * 2 + cid
        pltpu.sync_copy(dst_hbm.at[pl.ds(wid * NCHUNK, NCHUNK)], d_idx)
        for i in range(CHUNK // 16):
            ones_v[pl.ds(i * 16, 16)] = jnp.ones((16,), jnp.float32)
        pltpu.sync_copy(z_hbm.at[pl.ds(sid * ROWS_PT, ROWS_PT)],
                        acc.at[pl.ds(sid * ROWS_PT, ROWS_PT)])
        plsc.subcore_barrier()

        def body(j, c):
            pltpu.sync_copy(ones_v, acc.at[d_idx.at[j]], add=True)
            return c

        lax.fori_loop(0, NCHUNK, body, 0)
        plsc.subcore_barrier()
        pltpu.sync_copy(acc.at[pl.ds(sid * ROWS_PT, ROWS_PT)],
                        deg_out.at[pl.ds(cid * N_PAD + sid * ROWS_PT, ROWS_PT)])

    return run(dst_r, zeros1)


PC = NCHUNK // 2


def _sc_scatter(p, src_r, dst_r, zeros2):
    @functools.partial(
        pl.kernel,
        out_type=jax.ShapeDtypeStruct((2 * N_PAD, H), jnp.float32),
        mesh=_sc_mesh(),
        scratch_types=[
            pltpu.VMEM((PC, CHUNK), jnp.int32),
            pltpu.VMEM((PC, CHUNK), jnp.int32),
            pltpu.VMEM((CHUNK, H), jnp.float32),
            pltpu.VMEM((CHUNK, H), jnp.float32),
            pltpu.VMEM_SHARED((N_PAD, H), jnp.float32),
            pltpu.SemaphoreType.DMA,
            pltpu.SemaphoreType.DMA,
        ],
    )
    def run(p_hbm, src_hbm, dst_hbm, z_hbm, s_out,
            s_idx, d_idx, buf0, buf1, acc, sem0, sem1):
        cid = lax.axis_index("c")
        sid = lax.axis_index("s")
        wid = sid * 2 + cid
        pltpu.sync_copy(z_hbm.at[pl.ds(sid * ROWS_PT, ROWS_PT)],
                        acc.at[pl.ds(sid * ROWS_PT, ROWS_PT)])
        plsc.subcore_barrier()

        for ph in range(2):
            base = wid * NCHUNK + ph * PC
            pltpu.sync_copy(src_hbm.at[pl.ds(base, PC)], s_idx)
            pltpu.sync_copy(dst_hbm.at[pl.ds(base, PC)], d_idx)

            pltpu.async_copy(p_hbm.at[s_idx.at[0]], buf0, sem0)
            pltpu.async_copy(p_hbm.at[s_idx.at[1]], buf1, sem1)

            def body(g, c):
                j0 = 2 * g
                j1 = j0 + 1
                pltpu.make_async_copy(p_hbm.at[s_idx.at[j0]], buf0, sem0).wait()
                pltpu.sync_copy(buf0, acc.at[d_idx.at[j0]], add=True)

                @pl.when(j0 + 2 < PC)
                def _():
                    pltpu.async_copy(p_hbm.at[s_idx.at[j0 + 2]], buf0, sem0)

                pltpu.make_async_copy(p_hbm.at[s_idx.at[j1]], buf1, sem1).wait()
                pltpu.sync_copy(buf1, acc.at[d_idx.at[j1]], add=True)

                @pl.when(j1 + 2 < PC)
                def _():
                    pltpu.async_copy(p_hbm.at[s_idx.at[j1 + 2]], buf1, sem1)

                return c

            lax.fori_loop(0, PC // 2, body, 0)

        plsc.subcore_barrier()
        pltpu.sync_copy(acc.at[pl.ds(sid * ROWS_PT, ROWS_PT)],
                        s_out.at[pl.ds(cid * N_PAD + sid * ROWS_PT, ROWS_PT)])

    return run(p, src_r, dst_r, zeros2)


def _tc_first(dega, degb, x, W1):
    def body(dega_ref, degb_ref, x_ref, w_ref, p_ref, dinv_ref):
        deg = dega_ref[...] + degb_ref[...] + 1.0
        dinv = lax.rsqrt(deg)
        dinv_ref[...] = dinv
        p_ref[...] = dinv * jnp.dot(x_ref[...], w_ref[...],
                                    preferred_element_type=jnp.float32)

    nb = N_PAD // BR
    return pl.pallas_call(
        body,
        grid=(nb,),
        in_specs=[
            pl.BlockSpec((BR, 1), lambda i: (i, 0)),
            pl.BlockSpec((BR, 1), lambda i: (i, 0)),
            pl.BlockSpec((BR, D), lambda i: (i, 0)),
            pl.BlockSpec((D, H), lambda i: (0, 0)),
        ],
        out_specs=[
            pl.BlockSpec((BR, H), lambda i: (i, 0)),
            pl.BlockSpec((BR, 1), lambda i: (i, 0)),
        ],
        out_shape=[
            jax.ShapeDtypeStruct((N_PAD, H), jnp.float32),
            jax.ShapeDtypeStruct((N_PAD, 1), jnp.float32),
        ],
    )(dega, degb, x, W1)


def _tc_mid(sa, sb, pp, dinv, b2d, W):
    def body(sa_ref, sb_ref, pp_ref, dinv_ref, b_ref, w_ref, p_ref):
        dinv = dinv_ref[...]
        h = jnp.maximum(dinv * (sa_ref[...] + sb_ref[...] + pp_ref[...])
                        + b_ref[...], 0.0)
        p_ref[...] = dinv * jnp.dot(h, w_ref[...],
                                    preferred_element_type=jnp.float32)

    nb = N_PAD // BR
    return pl.pallas_call(
        body,
        grid=(nb,),
        in_specs=[
            pl.BlockSpec((BR, H), lambda i: (i, 0)),
            pl.BlockSpec((BR, H), lambda i: (i, 0)),
            pl.BlockSpec((BR, H), lambda i: (i, 0)),
            pl.BlockSpec((BR, 1), lambda i: (i, 0)),
            pl.BlockSpec((1, H), lambda i: (0, 0)),
            pl.BlockSpec((H, H), lambda i: (0, 0)),
        ],
        out_specs=pl.BlockSpec((BR, H), lambda i: (i, 0)),
        out_shape=jax.ShapeDtypeStruct((N_PAD, H), jnp.float32),
    )(sa, sb, pp, dinv, b2d, W)


def _tc_pool(sa, sb, p4, dinv, b2d, batch3):
    def body(sa_ref, sb_ref, p_ref, dinv_ref, b_ref, batch_ref,
             pooled_ref, counts_ref):
        i = pl.program_id(0)
        h = dinv_ref[...] * (sa_ref[...] + sb_ref[...] + p_ref[...]) + b_ref[...]
        bb = batch_ref[0, 0, :]
        seg = lax.broadcasted_iota(jnp.int32, (G_PAD, BRP), 0)
        oh = (seg == bb[None, :]).astype(jnp.float32)

        @pl.when(i == 0)
        def _():
            pooled_ref[...] = jnp.zeros_like(pooled_ref)
            counts_ref[...] = jnp.zeros_like(counts_ref)

        pooled_ref[...] += jnp.dot(oh, h, preferred_element_type=jnp.float32)
        counts_ref[...] += jnp.sum(oh, axis=1, keepdims=True)

    nb = N_PAD // BRP
    return pl.pallas_call(
        body,
        grid=(nb,),
        in_specs=[
            pl.BlockSpec((BRP, H), lambda i: (i, 0)),
            pl.BlockSpec((BRP, H), lambda i: (i, 0)),
            pl.BlockSpec((BRP, H), lambda i: (i, 0)),
            pl.BlockSpec((BRP, 1), lambda i: (i, 0)),
            pl.BlockSpec((1, H), lambda i: (0, 0)),
            pl.BlockSpec((1, 1, BRP), lambda i: (i, 0, 0)),
        ],
        out_specs=[
            pl.BlockSpec((G_PAD, H), lambda i: (0, 0)),
            pl.BlockSpec((G_PAD, 1), lambda i: (0, 0)),
        ],
        out_shape=[
            jax.ShapeDtypeStruct((G_PAD, H), jnp.float32),
            jax.ShapeDtypeStruct((G_PAD, 1), jnp.float32),
        ],
    )(sa, sb, p4, dinv, b2d, batch3)


def _tc_final(pooled, counts, Wl_pad, bl_pad):
    def body(pooled_ref, counts_ref, wl_ref, bl_ref, out_ref):
        avg = pooled_ref[...] / jnp.maximum(counts_ref[...], 1.0)
        out_ref[...] = jnp.dot(avg, wl_ref[...],
                               preferred_element_type=jnp.float32) + bl_ref[...]

    return pl.pallas_call(
        body,
        out_shape=jax.ShapeDtypeStruct((G_PAD, H), jnp.float32),
    )(pooled, counts, Wl_pad, bl_pad)


def kernel(x, edge_index, batch, W1, b1, W2, b2, W3, b3, W4, b4, Wl, bl):
    f32 = jnp.float32
    src = edge_index[0].astype(jnp.int32)
    dst = edge_index[1].astype(jnp.int32)
    pad_e = E_PAD - E
    pad_idx = N + jnp.arange(pad_e, dtype=jnp.int32) % (N_PAD - N)
    src_r = jnp.concatenate([src, pad_idx]).reshape(NW * NCHUNK, CHUNK)
    dst_r = jnp.concatenate([dst, pad_idx]).reshape(NW * NCHUNK, CHUNK)

    x_pad = jnp.concatenate([x.astype(f32), jnp.zeros((N_PAD - N, D), f32)])
    batch3 = jnp.concatenate([batch.astype(jnp.int32),
                              jnp.full((N_PAD - N,), G, jnp.int32)]
                             ).reshape(N_PAD // BRP, 1, BRP)
    zeros1 = jnp.zeros((N_PAD,), f32)
    zeros2 = jnp.zeros((N_PAD, H), f32)

    b1r = b1.reshape(1, H).astype(f32)
    b2r = b2.reshape(1, H).astype(f32)
    b3r = b3.reshape(1, H).astype(f32)
    b4r = b4.reshape(1, H).astype(f32)
    Wl_pad = jnp.pad(Wl.astype(f32), ((0, 0), (0, H - C)))
    bl_pad = jnp.pad(bl.astype(f32), (0, H - C)).reshape(1, H)

    deg_pair = _sc_degree(dst_r, zeros1).reshape(2, N_PAD)
    dega = deg_pair[0].reshape(N_PAD, 1)
    degb = deg_pair[1].reshape(N_PAD, 1)

    p1, dinv = _tc_first(dega, degb, x_pad, W1.astype(f32))
    s1 = _sc_scatter(p1, src_r, dst_r, zeros2).reshape(2, N_PAD, H)
    p2 = _tc_mid(s1[0], s1[1], p1, dinv, b1r, W2.astype(f32))
    s2 = _sc_scatter(p2, src_r, dst_r, zeros2).reshape(2, N_PAD, H)
    p3 = _tc_mid(s2[0], s2[1], p2, dinv, b2r, W3.astype(f32))
    s3 = _sc_scatter(p3, src_r, dst_r, zeros2).reshape(2, N_PAD, H)
    p4 = _tc_mid(s3[0], s3[1], p3, dinv, b3r, W4.astype(f32))
    s4 = _sc_scatter(p4, src_r, dst_r, zeros2).reshape(2, N_PAD, H)
    pooled, counts = _tc_pool(s4[0], s4[1], p4, dinv, b4r, batch3)
    out = _tc_final(pooled, counts, Wl_pad, bl_pad)
    return out[:G, :C]
